# baseline (speedup 1.0000x reference)
import functools

import jax
import jax.numpy as jnp
from jax import lax
from jax.experimental import pallas as pl
from jax.experimental.pallas import tpu as pltpu
from jax.experimental.pallas import tpu_sc as plsc

N = 10000
E = 320000
D = 128

NC = 2
NS = 16
NW = NC * NS
EPW = E // NW
B = 80
CH = EPW // B
NPAD = 10240
ROWS_PER_TILE = NPAD // NS


def _sc_body(x_hbm, src_hbm, dst_hbm, zero_hbm, out_hbm,
             acc, srcs, dsts, rows0, rows1, sem0, sem1):
    c = lax.axis_index("c")
    s = lax.axis_index("s")
    wid = s * NC + c

    r0 = pl.multiple_of(s * ROWS_PER_TILE, 8)
    z = pltpu.async_copy(zero_hbm, acc.at[pl.ds(r0, ROWS_PER_TILE)], sem0)
    i0 = pltpu.async_copy(src_hbm.at[0, wid], srcs, sem1)
    i1 = pltpu.async_copy(dst_hbm.at[1, wid], dsts, sem1)
    z.wait()
    i0.wait()
    i1.wait()

    plsc.subcore_barrier()

    bufs = (rows0, rows1)
    sems = (sem0, sem1)

    def gather(ci, buf, sem):
        pltpu.async_copy(x_hbm.at[srcs.at[pl.ds(ci * B, B)]], buf, sem)

    def wait_gather(ci, buf, sem):
        pltpu.make_async_copy(x_hbm.at[srcs.at[pl.ds(ci * B, B)]], buf,
                              sem).wait()

    gather(0, rows0, sem0)
    gather(1, rows1, sem1)

    def chunk_pair(ci0, carry):
        for b in range(2):
            ci = ci0 + b
            buf, sem = bufs[b], sems[b]
            wait_gather(ci, buf, sem)
            pltpu.sync_copy(buf, acc.at[dsts.at[ci]], add=True)
            @pl.when(ci + 2 < CH)
            def _():
                gather(ci + 2, buf, sem)
        return carry

    lax.fori_loop(0, (CH - 1) // 2, lambda i, cr: chunk_pair(i * 2, cr), 0)

    wait_gather(CH - 1, rows0, sem0)
    pltpu.sync_copy(rows0, acc.at[dsts.at[CH - 1]], add=True)

    plsc.subcore_barrier()

    pltpu.sync_copy(acc.at[pl.ds(r0, ROWS_PER_TILE)],
                    out_hbm.at[c, pl.ds(r0, ROWS_PER_TILE)])


_sc_aggregate = functools.partial(
    pl.kernel,
    out_type=jax.ShapeDtypeStruct((NC, NPAD, D), jnp.float32),
    mesh=plsc.VectorSubcoreMesh(core_axis_name="c", subcore_axis_name="s",
                                num_cores=NC, num_subcores=NS),
    scratch_types=[
        pltpu.VMEM_SHARED((NPAD, D), jnp.float32),
        pltpu.VMEM((EPW,), jnp.int32),
        pltpu.VMEM((CH, B), jnp.int32),
        pltpu.VMEM((B, D), jnp.float32),
        pltpu.VMEM((B, D), jnp.float32),
        pltpu.SemaphoreType.DMA,
        pltpu.SemaphoreType.DMA,
    ],
)(_sc_body)


def _v_body(x_ref, wvt_ref, bv_ref, o_ref):
    o_ref[...] = jnp.maximum(
        jnp.dot(x_ref[...], wvt_ref[...], preferred_element_type=jnp.float32)
        + bv_ref[...], 0.0)


def _tc_body(x_ref, v_ref, p_ref, wat_ref, ba_ref, o_ref):
    x = x_ref[...]
    aggr = p_ref[0] + p_ref[1]
    a = jnp.maximum(
        jnp.dot(aggr, wat_ref[...], preferred_element_type=jnp.float32)
        + ba_ref[...], 0.0)
    o_ref[...] = jnp.maximum(v_ref[...] + jnp.minimum(x, a), 0.0)


_TC_BLOCK = 2000


def _row_specs():
    row_spec = pl.BlockSpec((_TC_BLOCK, D), lambda i: (i, 0))
    full_spec = pl.BlockSpec((D, D), lambda i: (0, 0))
    bias_spec = pl.BlockSpec((1, D), lambda i: (0, 0))
    return row_spec, full_spec, bias_spec


def _v_pass(x, wvt, bv):
    row_spec, full_spec, bias_spec = _row_specs()
    return pl.pallas_call(
        _v_body,
        grid=(N // _TC_BLOCK,),
        in_specs=[row_spec, full_spec, bias_spec],
        out_specs=row_spec,
        out_shape=jax.ShapeDtypeStruct((N, D), jnp.float32),
    )(x, wvt, bv)


def _tc_combine(x, v, p, wat, ba):
    row_spec, full_spec, bias_spec = _row_specs()
    p_spec = pl.BlockSpec((NC, _TC_BLOCK, D), lambda i: (0, i, 0))
    return pl.pallas_call(
        _tc_body,
        grid=(N // _TC_BLOCK,),
        in_specs=[row_spec, row_spec, p_spec, full_spec, bias_spec],
        out_specs=row_spec,
        out_shape=jax.ShapeDtypeStruct((N, D), jnp.float32),
    )(x, v, p, wat, ba)


@jax.jit
def kernel(x, edge_index, batch, Wv, bv, Wa, ba):
    src = edge_index.reshape(2, NW, EPW)
    dst = edge_index.reshape(2, NW, CH, B)
    zeros = jnp.zeros((ROWS_PER_TILE, D), jnp.float32)
    partials = _sc_aggregate(x, src, dst, zeros)
    v = _v_pass(x, Wv.T, bv.reshape(1, D))
    h = _tc_combine(x, v, partials, Wa.T, ba.reshape(1, D))
    return h


# speedup vs baseline: 1.0071x; 1.0071x over previous
import functools

import jax
import jax.numpy as jnp
from jax import lax
from jax.experimental import pallas as pl
from jax.experimental.pallas import tpu as pltpu
from jax.experimental.pallas import tpu_sc as plsc

N = 10000
E = 320000
D = 128

NC = 2
NS = 16
NW = NC * NS
EPW = E // NW
B = 80
CH = EPW // B
NPAD = 10240
ROWS_PER_TILE = NPAD // NS


def _sc_body(x_hbm, src_hbm, dst_hbm, zero_hbm, out_hbm,
             acc, srcs, dsts, rows0, rows1, sem0, sem1, sem2):
    c = lax.axis_index("c")
    s = lax.axis_index("s")
    wid = s * NC + c

    r0 = pl.multiple_of(s * ROWS_PER_TILE, 8)
    z = pltpu.async_copy(zero_hbm, acc.at[pl.ds(r0, ROWS_PER_TILE)], sem2)
    i0 = pltpu.async_copy(src_hbm.at[0, wid], srcs, sem0)
    i1 = pltpu.async_copy(dst_hbm.at[1, wid], dsts, sem1)

    bufs = (rows0, rows1)
    sems = (sem0, sem1)

    def gather(ci, buf, sem):
        pltpu.async_copy(x_hbm.at[srcs.at[pl.ds(ci * B, B)]], buf, sem)

    def wait_gather(ci, buf, sem):
        pltpu.make_async_copy(x_hbm.at[srcs.at[pl.ds(ci * B, B)]], buf,
                              sem).wait()

    i0.wait()
    i1.wait()
    gather(0, rows0, sem0)
    gather(1, rows1, sem1)
    z.wait()

    plsc.subcore_barrier()

    def chunk_pair(ci0, carry):
        for b in range(2):
            ci = ci0 + b
            buf, sem = bufs[b], sems[b]
            wait_gather(ci, buf, sem)
            pltpu.sync_copy(buf, acc.at[dsts.at[ci]], add=True)
            @pl.when(ci + 2 < CH)
            def _():
                gather(ci + 2, buf, sem)
        return carry

    lax.fori_loop(0, (CH - 1) // 2, lambda i, cr: chunk_pair(i * 2, cr), 0)

    wait_gather(CH - 1, rows0, sem0)
    pltpu.sync_copy(rows0, acc.at[dsts.at[CH - 1]], add=True)

    plsc.subcore_barrier()

    pltpu.sync_copy(acc.at[pl.ds(r0, ROWS_PER_TILE)],
                    out_hbm.at[c, pl.ds(r0, ROWS_PER_TILE)])


_sc_aggregate = functools.partial(
    pl.kernel,
    out_type=jax.ShapeDtypeStruct((NC, NPAD, D), jnp.float32),
    mesh=plsc.VectorSubcoreMesh(core_axis_name="c", subcore_axis_name="s",
                                num_cores=NC, num_subcores=NS),
    scratch_types=[
        pltpu.VMEM_SHARED((NPAD, D), jnp.float32),
        pltpu.VMEM((EPW,), jnp.int32),
        pltpu.VMEM((CH, B), jnp.int32),
        pltpu.VMEM((B, D), jnp.float32),
        pltpu.VMEM((B, D), jnp.float32),
        pltpu.SemaphoreType.DMA,
        pltpu.SemaphoreType.DMA,
        pltpu.SemaphoreType.DMA,
    ],
)(_sc_body)


def _v_body(x_ref, wvt_ref, bv_ref, o_ref):
    o_ref[...] = jnp.maximum(
        jnp.dot(x_ref[...], wvt_ref[...], preferred_element_type=jnp.float32)
        + bv_ref[...], 0.0)


def _tc_body(x_ref, v_ref, p_ref, wat_ref, ba_ref, o_ref):
    x = x_ref[...]
    aggr = p_ref[0] + p_ref[1]
    a = jnp.maximum(
        jnp.dot(aggr, wat_ref[...], preferred_element_type=jnp.float32)
        + ba_ref[...], 0.0)
    o_ref[...] = jnp.maximum(v_ref[...] + jnp.minimum(x, a), 0.0)


_TC_BLOCK = 2000


def _row_specs():
    row_spec = pl.BlockSpec((_TC_BLOCK, D), lambda i: (i, 0))
    full_spec = pl.BlockSpec((D, D), lambda i: (0, 0))
    bias_spec = pl.BlockSpec((1, D), lambda i: (0, 0))
    return row_spec, full_spec, bias_spec


def _v_pass(x, wvt, bv):
    row_spec, full_spec, bias_spec = _row_specs()
    return pl.pallas_call(
        _v_body,
        grid=(N // _TC_BLOCK,),
        in_specs=[row_spec, full_spec, bias_spec],
        out_specs=row_spec,
        out_shape=jax.ShapeDtypeStruct((N, D), jnp.float32),
    )(x, wvt, bv)


def _tc_combine(x, v, p, wat, ba):
    row_spec, full_spec, bias_spec = _row_specs()
    p_spec = pl.BlockSpec((NC, _TC_BLOCK, D), lambda i: (0, i, 0))
    return pl.pallas_call(
        _tc_body,
        grid=(N // _TC_BLOCK,),
        in_specs=[row_spec, row_spec, p_spec, full_spec, bias_spec],
        out_specs=row_spec,
        out_shape=jax.ShapeDtypeStruct((N, D), jnp.float32),
    )(x, v, p, wat, ba)


@jax.jit
def kernel(x, edge_index, batch, Wv, bv, Wa, ba):
    src = edge_index.reshape(2, NW, EPW)
    dst = edge_index.reshape(2, NW, CH, B)
    zeros = jnp.zeros((ROWS_PER_TILE, D), jnp.float32)
    v = _v_pass(x, Wv.T, bv.reshape(1, D))
    partials = _sc_aggregate(x, src, dst, zeros)
    h = _tc_combine(x, v, partials, Wa.T, ba.reshape(1, D))
    return h


# speedup vs baseline: 1.0285x; 1.0213x over previous
import functools

import jax
import jax.numpy as jnp
from jax import lax
from jax.experimental import pallas as pl
from jax.experimental.pallas import tpu as pltpu
from jax.experimental.pallas import tpu_sc as plsc

N = 10000
E = 320000
D = 128

NC = 2
NS = 16
NW = NC * NS
EPW = E // NW
B = 80
CH = EPW // B
NPAD = 10240
ROWS_PER_TILE = NPAD // NS


def _sc_body(x_hbm, src_hbm, dst_hbm, zero_hbm, out_hbm,
             acc, srcs, dsts, rows0, rows1, sem0, sem1, sem2):
    c = lax.axis_index("c")
    s = lax.axis_index("s")
    wid = s * NC + c

    r0 = pl.multiple_of(s * ROWS_PER_TILE, 8)
    z = pltpu.async_copy(zero_hbm, acc.at[pl.ds(r0, ROWS_PER_TILE)], sem2)
    i0 = pltpu.async_copy(src_hbm.at[0, wid], srcs, sem0)
    i1 = pltpu.async_copy(dst_hbm.at[1, wid], dsts, sem1)

    bufs = (rows0, rows1)
    sems = (sem0, sem1)

    def gather(ci, buf, sem):
        pltpu.async_copy(x_hbm.at[srcs.at[pl.ds(ci * B, B)]], buf, sem)

    def wait_gather(ci, buf, sem):
        pltpu.make_async_copy(x_hbm.at[srcs.at[pl.ds(ci * B, B)]], buf,
                              sem).wait()

    i0.wait()
    i1.wait()
    gather(0, rows0, sem0)
    gather(1, rows1, sem1)
    z.wait()

    plsc.subcore_barrier()

    def chunk_pair(ci0, carry):
        for b in range(2):
            ci = ci0 + b
            buf, sem = bufs[b], sems[b]
            wait_gather(ci, buf, sem)
            pltpu.sync_copy(buf, acc.at[dsts.at[ci]], add=True)
            @pl.when(ci + 2 < CH)
            def _():
                gather(ci + 2, buf, sem)
        return carry

    lax.fori_loop(0, (CH - 1) // 2, lambda i, cr: chunk_pair(i * 2, cr), 0)

    wait_gather(CH - 1, rows0, sem0)
    pltpu.sync_copy(rows0, acc.at[dsts.at[CH - 1]], add=True)

    plsc.subcore_barrier()

    pltpu.sync_copy(acc.at[pl.ds(r0, ROWS_PER_TILE)],
                    out_hbm.at[c, pl.ds(r0, ROWS_PER_TILE)])


_sc_aggregate = functools.partial(
    pl.kernel,
    out_type=jax.ShapeDtypeStruct((NC, NPAD, D), jnp.float32),
    mesh=plsc.VectorSubcoreMesh(core_axis_name="c", subcore_axis_name="s",
                                num_cores=NC, num_subcores=NS),
    scratch_types=[
        pltpu.VMEM_SHARED((NPAD, D), jnp.float32),
        pltpu.VMEM((EPW,), jnp.int32),
        pltpu.VMEM((CH, B), jnp.int32),
        pltpu.VMEM((B, D), jnp.float32),
        pltpu.VMEM((B, D), jnp.float32),
        pltpu.SemaphoreType.DMA,
        pltpu.SemaphoreType.DMA,
        pltpu.SemaphoreType.DMA,
    ],
)(_sc_body)


def _tc_body(x_ref, p_ref, wvt_ref, bv_ref, wat_ref, ba_ref, o_ref):
    x = x_ref[...]
    aggr = p_ref[0] + p_ref[1]
    v = jnp.maximum(
        jnp.dot(x, wvt_ref[...], preferred_element_type=jnp.float32)
        + bv_ref[...], 0.0)
    a = jnp.maximum(
        jnp.dot(aggr, wat_ref[...], preferred_element_type=jnp.float32)
        + ba_ref[...], 0.0)
    o_ref[...] = jnp.maximum(v + jnp.minimum(x, a), 0.0)


_TC_BLOCK = 2000


def _tc_combine(x, p, wvt, bv, wat, ba):
    row_spec = pl.BlockSpec((_TC_BLOCK, D), lambda i: (i, 0))
    p_spec = pl.BlockSpec((NC, _TC_BLOCK, D), lambda i: (0, i, 0))
    full_spec = pl.BlockSpec((D, D), lambda i: (0, 0))
    bias_spec = pl.BlockSpec((1, D), lambda i: (0, 0))
    return pl.pallas_call(
        _tc_body,
        grid=(N // _TC_BLOCK,),
        in_specs=[row_spec, p_spec, full_spec, bias_spec, full_spec,
                  bias_spec],
        out_specs=row_spec,
        out_shape=jax.ShapeDtypeStruct((N, D), jnp.float32),
    )(x, p, wvt, bv, wat, ba)


@jax.jit
def kernel(x, edge_index, batch, Wv, bv, Wa, ba):
    src = edge_index.reshape(2, NW, EPW)
    dst = edge_index.reshape(2, NW, CH, B)
    zeros = jnp.zeros((ROWS_PER_TILE, D), jnp.float32)
    partials = _sc_aggregate(x, src, dst, zeros)
    h = _tc_combine(x, partials,
                    Wv.T, bv.reshape(1, D), Wa.T, ba.reshape(1, D))
    return h


# speedup vs baseline: 1.0293x; 1.0007x over previous
import functools

import jax
import jax.numpy as jnp
from jax import lax
from jax.experimental import pallas as pl
from jax.experimental.pallas import tpu as pltpu
from jax.experimental.pallas import tpu_sc as plsc

N = 10000
E = 320000
D = 128

NC = 2
NS = 16
NW = NC * NS
EPW = E // NW
B = 80
CH = EPW // B
NPAD = 10240
ROWS_PER_TILE = NPAD // NS


def _sc_body(x_hbm, src_hbm, dst_hbm, zero_hbm, out_hbm,
             acc, srcs, dsts, rows0, rows1, sem0, sem1, sem2):
    c = lax.axis_index("c")
    s = lax.axis_index("s")
    wid = s * NC + c

    r0 = pl.multiple_of(s * ROWS_PER_TILE, 8)
    z = pltpu.async_copy(zero_hbm, acc.at[pl.ds(r0, ROWS_PER_TILE)], sem2)
    i0 = pltpu.async_copy(src_hbm.at[0, wid], srcs, sem0)
    i1 = pltpu.async_copy(dst_hbm.at[1, wid], dsts, sem1)

    bufs = (rows0, rows1)
    sems = (sem0, sem1)

    H = B // 2

    def gather(ci, buf, sem):
        pltpu.async_copy(x_hbm.at[srcs.at[pl.ds(ci * B, H)]],
                         buf.at[pl.ds(0, H)], sem)
        pltpu.async_copy(x_hbm.at[srcs.at[pl.ds(ci * B + H, H)]],
                         buf.at[pl.ds(H, H)], sem)

    def wait_gather(ci, buf, sem):
        pltpu.make_async_copy(x_hbm.at[srcs.at[pl.ds(ci * B, H)]],
                              buf.at[pl.ds(0, H)], sem).wait()
        pltpu.make_async_copy(x_hbm.at[srcs.at[pl.ds(ci * B + H, H)]],
                              buf.at[pl.ds(H, H)], sem).wait()

    i0.wait()
    i1.wait()
    gather(0, rows0, sem0)
    gather(1, rows1, sem1)
    z.wait()

    plsc.subcore_barrier()

    def chunk_pair(ci0, carry):
        for b in range(2):
            ci = ci0 + b
            buf, sem = bufs[b], sems[b]
            wait_gather(ci, buf, sem)
            pltpu.sync_copy(buf, acc.at[dsts.at[ci]], add=True)
            @pl.when(ci + 2 < CH)
            def _():
                gather(ci + 2, buf, sem)
        return carry

    lax.fori_loop(0, (CH - 1) // 2, lambda i, cr: chunk_pair(i * 2, cr), 0)

    wait_gather(CH - 1, rows0, sem0)
    pltpu.sync_copy(rows0, acc.at[dsts.at[CH - 1]], add=True)

    plsc.subcore_barrier()

    pltpu.sync_copy(acc.at[pl.ds(r0, ROWS_PER_TILE)],
                    out_hbm.at[c, pl.ds(r0, ROWS_PER_TILE)])


_sc_aggregate = functools.partial(
    pl.kernel,
    out_type=jax.ShapeDtypeStruct((NC, NPAD, D), jnp.float32),
    mesh=plsc.VectorSubcoreMesh(core_axis_name="c", subcore_axis_name="s",
                                num_cores=NC, num_subcores=NS),
    scratch_types=[
        pltpu.VMEM_SHARED((NPAD, D), jnp.float32),
        pltpu.VMEM((EPW,), jnp.int32),
        pltpu.VMEM((CH, B), jnp.int32),
        pltpu.VMEM((B, D), jnp.float32),
        pltpu.VMEM((B, D), jnp.float32),
        pltpu.SemaphoreType.DMA,
        pltpu.SemaphoreType.DMA,
        pltpu.SemaphoreType.DMA,
    ],
)(_sc_body)


def _tc_body(x_ref, p_ref, wvt_ref, bv_ref, wat_ref, ba_ref, o_ref):
    x = x_ref[...]
    aggr = p_ref[0] + p_ref[1]
    v = jnp.maximum(
        jnp.dot(x, wvt_ref[...], preferred_element_type=jnp.float32)
        + bv_ref[...], 0.0)
    a = jnp.maximum(
        jnp.dot(aggr, wat_ref[...], preferred_element_type=jnp.float32)
        + ba_ref[...], 0.0)
    o_ref[...] = jnp.maximum(v + jnp.minimum(x, a), 0.0)


_TC_BLOCK = 2000


def _tc_combine(x, p, wvt, bv, wat, ba):
    row_spec = pl.BlockSpec((_TC_BLOCK, D), lambda i: (i, 0))
    p_spec = pl.BlockSpec((NC, _TC_BLOCK, D), lambda i: (0, i, 0))
    full_spec = pl.BlockSpec((D, D), lambda i: (0, 0))
    bias_spec = pl.BlockSpec((1, D), lambda i: (0, 0))
    return pl.pallas_call(
        _tc_body,
        grid=(N // _TC_BLOCK,),
        in_specs=[row_spec, p_spec, full_spec, bias_spec, full_spec,
                  bias_spec],
        out_specs=row_spec,
        out_shape=jax.ShapeDtypeStruct((N, D), jnp.float32),
    )(x, p, wvt, bv, wat, ba)


@jax.jit
def kernel(x, edge_index, batch, Wv, bv, Wa, ba):
    src = edge_index.reshape(2, NW, EPW)
    dst = edge_index.reshape(2, NW, CH, B)
    zeros = jnp.zeros((ROWS_PER_TILE, D), jnp.float32)
    partials = _sc_aggregate(x, src, dst, zeros)
    h = _tc_combine(x, partials,
                    Wv.T, bv.reshape(1, D), Wa.T, ba.reshape(1, D))
    return h


# speedup vs baseline: 1.0788x; 1.0481x over previous
import functools

import jax
import jax.numpy as jnp
from jax import lax
from jax.experimental import pallas as pl
from jax.experimental.pallas import tpu as pltpu
from jax.experimental.pallas import tpu_sc as plsc

N = 10000
E = 320000
D = 128

NC = 2
NS = 16
NW = NC * NS
EPW = E // NW
B = 80
CH = EPW // B
NPAD = 10240
ROWS_PER_TILE = NPAD // NS


def _sc_body(x_hbm, src_hbm, dst_hbm, out_hbm,
             acc, srcs, dsts, rows0, rows1, sem0, sem1, sem2):
    c = lax.axis_index("c")
    s = lax.axis_index("s")
    wid = s * NC + c

    i0 = pltpu.async_copy(src_hbm.at[0, wid], srcs, sem0)
    i1 = pltpu.async_copy(dst_hbm.at[1, wid], dsts, sem1)

    zvec = jnp.zeros((16,), jnp.float32)

    def zrow(r, carry):
        for j in range(D // 16):
            rows1[r, pl.ds(j * 16, 16)] = zvec
        return carry

    lax.fori_loop(0, B, zrow, 0)
    r0 = pl.multiple_of(s * ROWS_PER_TILE, 8)
    for k in range(ROWS_PER_TILE // B):
        pltpu.async_copy(rows1, acc.at[pl.ds(r0 + k * B, B)], sem2)

    bufs = (rows0, rows1)
    sems = (sem0, sem1)

    def gather(ci, buf, sem):
        pltpu.async_copy(x_hbm.at[srcs.at[pl.ds(ci * B, B)]], buf, sem)

    def wait_gather(ci, buf, sem):
        pltpu.make_async_copy(x_hbm.at[srcs.at[pl.ds(ci * B, B)]], buf,
                              sem).wait()

    i0.wait()
    gather(0, rows0, sem0)
    i1.wait()
    for k in range(ROWS_PER_TILE // B):
        pltpu.make_async_copy(rows1, acc.at[pl.ds(r0 + k * B, B)],
                              sem2).wait()
    gather(1, rows1, sem1)

    plsc.subcore_barrier()

    def chunk_pair(ci0, carry):
        for b in range(2):
            ci = ci0 + b
            buf, sem = bufs[b], sems[b]
            wait_gather(ci, buf, sem)
            pltpu.sync_copy(buf, acc.at[dsts.at[ci]], add=True)
            @pl.when(ci + 2 < CH)
            def _():
                gather(ci + 2, buf, sem)
        return carry

    lax.fori_loop(0, (CH - 1) // 2, lambda i, cr: chunk_pair(i * 2, cr), 0)

    wait_gather(CH - 1, rows0, sem0)
    pltpu.sync_copy(rows0, acc.at[dsts.at[CH - 1]], add=True)

    plsc.subcore_barrier()

    pltpu.sync_copy(acc.at[pl.ds(r0, ROWS_PER_TILE)],
                    out_hbm.at[c, pl.ds(r0, ROWS_PER_TILE)])


_sc_aggregate = functools.partial(
    pl.kernel,
    out_type=jax.ShapeDtypeStruct((NC, NPAD, D), jnp.float32),
    mesh=plsc.VectorSubcoreMesh(core_axis_name="c", subcore_axis_name="s",
                                num_cores=NC, num_subcores=NS),
    scratch_types=[
        pltpu.VMEM_SHARED((NPAD, D), jnp.float32),
        pltpu.VMEM((EPW,), jnp.int32),
        pltpu.VMEM((CH, B), jnp.int32),
        pltpu.VMEM((B, D), jnp.float32),
        pltpu.VMEM((B, D), jnp.float32),
        pltpu.SemaphoreType.DMA,
        pltpu.SemaphoreType.DMA,
        pltpu.SemaphoreType.DMA,
    ],
)(_sc_body)


def _tc_body(x_ref, p_ref, wvt_ref, bv_ref, wat_ref, ba_ref, o_ref):
    x = x_ref[...]
    aggr = p_ref[0] + p_ref[1]
    v = jnp.maximum(
        jnp.dot(x, wvt_ref[...], preferred_element_type=jnp.float32)
        + bv_ref[...], 0.0)
    a = jnp.maximum(
        jnp.dot(aggr, wat_ref[...], preferred_element_type=jnp.float32)
        + ba_ref[...], 0.0)
    o_ref[...] = jnp.maximum(v + jnp.minimum(x, a), 0.0)


_TC_BLOCK = 2000


def _tc_combine(x, p, wvt, bv, wat, ba):
    row_spec = pl.BlockSpec((_TC_BLOCK, D), lambda i: (i, 0))
    p_spec = pl.BlockSpec((NC, _TC_BLOCK, D), lambda i: (0, i, 0))
    full_spec = pl.BlockSpec((D, D), lambda i: (0, 0))
    bias_spec = pl.BlockSpec((1, D), lambda i: (0, 0))
    return pl.pallas_call(
        _tc_body,
        grid=(N // _TC_BLOCK,),
        in_specs=[row_spec, p_spec, full_spec, bias_spec, full_spec,
                  bias_spec],
        out_specs=row_spec,
        out_shape=jax.ShapeDtypeStruct((N, D), jnp.float32),
    )(x, p, wvt, bv, wat, ba)


@jax.jit
def kernel(x, edge_index, batch, Wv, bv, Wa, ba):
    src = edge_index.reshape(2, NW, EPW)
    dst = edge_index.reshape(2, NW, CH, B)
    partials = _sc_aggregate(x, src, dst)
    h = _tc_combine(x, partials,
                    Wv.T, bv.reshape(1, D), Wa.T, ba.reshape(1, D))
    return h


# speedup vs baseline: 1.0809x; 1.0019x over previous
import functools

import jax
import jax.numpy as jnp
from jax import lax
from jax.experimental import pallas as pl
from jax.experimental.pallas import tpu as pltpu
from jax.experimental.pallas import tpu_sc as plsc

N = 10000
E = 320000
D = 128

NC = 2
NS = 16
NW = NC * NS
EPW = E // NW
B = 80
CH = EPW // B
NPAD = 10240
ROWS_PER_TILE = NPAD // NS


ZR = 16


def _sc_body(x_hbm, src_hbm, dst_hbm, out_hbm,
             acc, srcs, dsts, rows0, rows1, zbuf, sem0, sem1, sem2):
    c = lax.axis_index("c")
    s = lax.axis_index("s")
    wid = s * NC + c

    i0 = pltpu.async_copy(src_hbm.at[0, wid], srcs, sem0)
    i1 = pltpu.async_copy(dst_hbm.at[1, wid], dsts, sem1)

    zvec = jnp.zeros((16,), jnp.float32)

    def zrow(r, carry):
        for j in range(D // 16):
            zbuf[r, pl.ds(j * 16, 16)] = zvec
        return carry

    lax.fori_loop(0, ZR, zrow, 0)
    r0 = pl.multiple_of(s * ROWS_PER_TILE, 8)
    for k in range(ROWS_PER_TILE // ZR):
        pltpu.async_copy(zbuf, acc.at[pl.ds(r0 + k * ZR, ZR)], sem2)

    bufs = (rows0, rows1)
    sems = (sem0, sem1)

    def gather(ci, buf, sem):
        pltpu.async_copy(x_hbm.at[srcs.at[pl.ds(ci * B, B)]], buf, sem)

    def wait_gather(ci, buf, sem):
        pltpu.make_async_copy(x_hbm.at[srcs.at[pl.ds(ci * B, B)]], buf,
                              sem).wait()

    i0.wait()
    gather(0, rows0, sem0)
    i1.wait()
    gather(1, rows1, sem1)
    for k in range(ROWS_PER_TILE // ZR):
        pltpu.make_async_copy(zbuf, acc.at[pl.ds(r0 + k * ZR, ZR)],
                              sem2).wait()

    plsc.subcore_barrier()

    def chunk_pair(ci0, carry):
        for b in range(2):
            ci = ci0 + b
            buf, sem = bufs[b], sems[b]
            wait_gather(ci, buf, sem)
            pltpu.sync_copy(buf, acc.at[dsts.at[ci]], add=True)
            @pl.when(ci + 2 < CH)
            def _():
                gather(ci + 2, buf, sem)
        return carry

    lax.fori_loop(0, (CH - 1) // 2, lambda i, cr: chunk_pair(i * 2, cr), 0)

    wait_gather(CH - 1, rows0, sem0)
    pltpu.sync_copy(rows0, acc.at[dsts.at[CH - 1]], add=True)

    plsc.subcore_barrier()

    pltpu.sync_copy(acc.at[pl.ds(r0, ROWS_PER_TILE)],
                    out_hbm.at[c, pl.ds(r0, ROWS_PER_TILE)])


_sc_aggregate = functools.partial(
    pl.kernel,
    out_type=jax.ShapeDtypeStruct((NC, NPAD, D), jnp.float32),
    mesh=plsc.VectorSubcoreMesh(core_axis_name="c", subcore_axis_name="s",
                                num_cores=NC, num_subcores=NS),
    scratch_types=[
        pltpu.VMEM_SHARED((NPAD, D), jnp.float32),
        pltpu.VMEM((EPW,), jnp.int32),
        pltpu.VMEM((CH, B), jnp.int32),
        pltpu.VMEM((B, D), jnp.float32),
        pltpu.VMEM((B, D), jnp.float32),
        pltpu.VMEM((ZR, D), jnp.float32),
        pltpu.SemaphoreType.DMA,
        pltpu.SemaphoreType.DMA,
        pltpu.SemaphoreType.DMA,
    ],
)(_sc_body)


def _tc_body(x_ref, p_ref, wvt_ref, bv_ref, wat_ref, ba_ref, o_ref):
    x = x_ref[...]
    aggr = p_ref[0] + p_ref[1]
    v = jnp.maximum(
        jnp.dot(x, wvt_ref[...], preferred_element_type=jnp.float32)
        + bv_ref[...], 0.0)
    a = jnp.maximum(
        jnp.dot(aggr, wat_ref[...], preferred_element_type=jnp.float32)
        + ba_ref[...], 0.0)
    o_ref[...] = jnp.maximum(v + jnp.minimum(x, a), 0.0)


_TC_BLOCK = 2000


def _tc_combine(x, p, wvt, bv, wat, ba):
    row_spec = pl.BlockSpec((_TC_BLOCK, D), lambda i: (i, 0))
    p_spec = pl.BlockSpec((NC, _TC_BLOCK, D), lambda i: (0, i, 0))
    full_spec = pl.BlockSpec((D, D), lambda i: (0, 0))
    bias_spec = pl.BlockSpec((1, D), lambda i: (0, 0))
    return pl.pallas_call(
        _tc_body,
        grid=(N // _TC_BLOCK,),
        in_specs=[row_spec, p_spec, full_spec, bias_spec, full_spec,
                  bias_spec],
        out_specs=row_spec,
        out_shape=jax.ShapeDtypeStruct((N, D), jnp.float32),
    )(x, p, wvt, bv, wat, ba)


@jax.jit
def kernel(x, edge_index, batch, Wv, bv, Wa, ba):
    src = edge_index.reshape(2, NW, EPW)
    dst = edge_index.reshape(2, NW, CH, B)
    partials = _sc_aggregate(x, src, dst)
    h = _tc_combine(x, partials,
                    Wv.T, bv.reshape(1, D), Wa.T, ba.reshape(1, D))
    return h
